# baseline (speedup 1.0000x reference)
import jax
import jax.numpy as jnp
from jax import lax
from jax.experimental import pallas as pl
from jax.experimental.pallas import tpu as pltpu
from jax.experimental.pallas import tpu_sc as plsc

N = 10000
E = 320000
C = 128
NPAD = 10240
RPT = 640
K = 80
R = 1000
EPW = E // 32
NBLK = EPW // 80

_mesh = plsc.VectorSubcoreMesh(core_axis_name="c", subcore_axis_name="s")
f32 = jnp.float32


def _fill_vec(ref, n, val):
    def body(j, _):
        ref[pl.ds(j * 16, 16)] = jnp.full((16,), val, f32)
        return 0
    lax.fori_loop(0, n // 16, body, 0)


NR = NPAD // C


def _zero_2d(ref, rows):
    def body(r, _):
        for c4 in range(C // 16):
            ref[r, pl.ds(c4 * 16, 16)] = jnp.zeros((16,), f32)
        return 0
    lax.fori_loop(0, rows, body, 0)


def _fill_iota(ref, n):
    def body(j, _):
        ref[pl.ds(j * 16, 16)] = jnp.arange(16, dtype=jnp.int32) + j * 16
        return 0
    lax.fori_loop(0, n // 16, body, 0)


def _drain_and_writeback(acc_l, acc_s, idt, out_hbm, cid, sid, wbuf):
    pltpu.sync_copy(acc_l, acc_s.at[idt], add=True)
    plsc.subcore_barrier()
    rows = NR // 16
    pltpu.sync_copy(acc_s.at[pl.ds(sid * rows, rows)], wbuf)
    pltpu.sync_copy(wbuf, out_hbm.at[cid, pl.ds(sid * rows, rows)])


def _deg_body(dst_hbm, d_hbm, acc_s, acc_l, idt, dbig, wbuf):
    cid = lax.axis_index("c")
    sid = lax.axis_index("s")
    _zero_2d(acc_l, NR)
    _fill_iota(idt, NR)
    rows = NR // 16
    pltpu.sync_copy(acc_l.at[pl.ds(0, rows)], acc_s.at[pl.ds(sid * rows, rows)])
    wid = sid * 2 + cid
    pltpu.sync_copy(dst_hbm.at[pl.ds(wid * NBLK, NBLK)], dbig)
    plsc.subcore_barrier()

    ones16 = jnp.ones((16,), f32)

    def ebody(i, _):
        for j in range(K // 16):
            d16 = dbig[i, pl.ds(j * 16, 16)]
            row = lax.shift_right_logical(d16, 7)
            col = jnp.bitwise_and(d16, 127)
            plsc.addupdate_scatter(acc_l, [row, col], ones16)
        return 0
    lax.fori_loop(0, NBLK, ebody, 0)
    plsc.subcore_barrier()
    _drain_and_writeback(acc_l, acc_s, idt, d_hbm, cid, sid, wbuf)


_deg_call = pl.kernel(
    _deg_body,
    out_type=jax.ShapeDtypeStruct((2, NR, C), f32),
    mesh=_mesh,
    compiler_params=pltpu.CompilerParams(use_tc_tiling_on_sc=False, needs_layout_passes=False),
    scratch_types=[
        pltpu.VMEM_SHARED((NR, C), f32),
        pltpu.VMEM((NR, C), f32),
        pltpu.VMEM((NR,), jnp.int32),
        pltpu.VMEM((NBLK, K), jnp.int32),
        pltpu.VMEM((NR // 16, C), f32),
    ],
)


CH = 25


def _edge_body(g_hbm, src_hbm, dst_hbm, p_hbm, acc_s,
               buf0, buf1, sbig, dbig, sem0, sem1, csem0, csem1):
    cid = lax.axis_index("c")
    sid = lax.axis_index("s")

    def zrow(r, _):
        for c4 in range(C // 16):
            buf0[r, pl.ds(c4 * 16, 16)] = jnp.zeros((16,), f32)
        return 0
    lax.fori_loop(0, K, zrow, 0)

    def zcp(k, _):
        pltpu.sync_copy(buf0, acc_s.at[pl.ds(sid * RPT + k * K, K)])
        return 0
    lax.fori_loop(0, RPT // K, zcp, 0)
    plsc.subcore_barrier()

    wid = sid * 2 + cid

    def gstart(b, bf, gs):
        pltpu.async_copy(g_hbm.at[sbig.at[b]], bf, gs)

    def gwait(b, bf, gs):
        pltpu.make_async_copy(g_hbm.at[sbig.at[b]], bf, gs).wait()

    def sstart(b, bf, cs):
        pltpu.async_copy(bf, acc_s.at[dbig.at[b]], cs, add=True)

    def swait(b, bf, cs):
        pltpu.make_async_copy(bf, acc_s.at[dbig.at[b]], cs).wait()

    def chunk(c, _):
        rowbase = wid * NBLK + c * CH
        pltpu.sync_copy(src_hbm.at[pl.ds(rowbase, CH)], sbig)
        pltpu.sync_copy(dst_hbm.at[pl.ds(rowbase, CH)], dbig)
        gstart(0, buf0, sem0)
        gstart(1, buf1, sem1)

        def step(o, _):
            t = 2 * o
            gwait(t, buf0, sem0)
            sstart(t, buf0, csem0)
            gwait(t + 1, buf1, sem1)
            sstart(t + 1, buf1, csem1)
            swait(t, buf0, csem0)

            @pl.when(t + 2 < CH)
            def _():
                gstart(t + 2, buf0, sem0)
            swait(t + 1, buf1, csem1)

            @pl.when(t + 3 < CH)
            def _():
                gstart(t + 3, buf1, sem1)
            return 0
        lax.fori_loop(0, CH // 2, step, 0)
        gwait(CH - 1, buf0, sem0)
        sstart(CH - 1, buf0, csem0)
        swait(CH - 1, buf0, csem0)
        return 0
    lax.fori_loop(0, NBLK // CH, chunk, 0)
    plsc.subcore_barrier()

    def wb(k, _):
        base = sid * RPT + k * K
        pltpu.sync_copy(acc_s.at[pl.ds(base, K)], buf0)
        pltpu.sync_copy(buf0, p_hbm.at[cid, pl.ds(base, K)])
        return 0
    lax.fori_loop(0, RPT // K, wb, 0)


_edge_call = pl.kernel(
    _edge_body,
    out_type=jax.ShapeDtypeStruct((2, NPAD, C), f32),
    mesh=_mesh,
    compiler_params=pltpu.CompilerParams(use_tc_tiling_on_sc=False, needs_layout_passes=False),
    scratch_types=[
        pltpu.VMEM_SHARED((NPAD, C), f32),
        pltpu.VMEM((K, C), f32),
        pltpu.VMEM((K, C), f32),
        pltpu.VMEM((CH, K), jnp.int32),
        pltpu.VMEM((CH, K), jnp.int32),
        pltpu.SemaphoreType.DMA,
        pltpu.SemaphoreType.DMA,
        pltpu.SemaphoreType.DMA,
        pltpu.SemaphoreType.DMA,
    ],
)


def _fin_body(gf_hbm, src_hbm, dst_hbm, a_hbm,
              acc_s, acc_l, gf_v, idt, sbig, dbig, wbuf):
    cid = lax.axis_index("c")
    sid = lax.axis_index("s")
    _zero_2d(acc_l, NR)
    _fill_iota(idt, NR)
    rows = NR // 16
    pltpu.sync_copy(acc_l.at[pl.ds(0, rows)], acc_s.at[pl.ds(sid * rows, rows)])
    pltpu.sync_copy(gf_hbm, gf_v)
    wid = sid * 2 + cid
    pltpu.sync_copy(src_hbm.at[pl.ds(wid * NBLK, NBLK)], sbig)
    pltpu.sync_copy(dst_hbm.at[pl.ds(wid * NBLK, NBLK)], dbig)
    plsc.subcore_barrier()

    def ebody(i, _):
        for j in range(K // 16):
            s16 = sbig[i, pl.ds(j * 16, 16)]
            d16 = dbig[i, pl.ds(j * 16, 16)]
            vals = plsc.load_gather(gf_v, [s16])
            row = lax.shift_right_logical(d16, 7)
            col = jnp.bitwise_and(d16, 127)
            plsc.addupdate_scatter(acc_l, [row, col], vals)
        return 0
    lax.fori_loop(0, NBLK, ebody, 0)
    plsc.subcore_barrier()
    _drain_and_writeback(acc_l, acc_s, idt, a_hbm, cid, sid, wbuf)


_fin_call = pl.kernel(
    _fin_body,
    out_type=jax.ShapeDtypeStruct((2, NR, C), f32),
    mesh=_mesh,
    compiler_params=pltpu.CompilerParams(use_tc_tiling_on_sc=False, needs_layout_passes=False),
    scratch_types=[
        pltpu.VMEM_SHARED((NR, C), f32),
        pltpu.VMEM((NR, C), f32),
        pltpu.VMEM((N,), f32),
        pltpu.VMEM((NR,), jnp.int32),
        pltpu.VMEM((NBLK, K), jnp.int32),
        pltpu.VMEM((NBLK, K), jnp.int32),
        pltpu.VMEM((NR // 16, C), f32),
    ],
)


def _tc1_body(x_ref, w_ref, d0_ref, d1_ref, g_ref, dinv_ref):
    dinv = lax.rsqrt(d0_ref[0] + d1_ref[0] + 1.0)
    g_ref[...] = jnp.dot(x_ref[...], w_ref[...],
                         preferred_element_type=f32) * dinv
    dinv_ref[...] = dinv


def _tc2_body(p0_ref, p1_ref, g_ref, dinv_ref, b_ref, w_ref, o_ref):
    dinv = dinv_ref[...]
    h = jnp.maximum(
        dinv * (p0_ref[0] + p1_ref[0] + g_ref[...]) + b_ref[...], 0.0)
    o_ref[...] = jnp.dot(h, w_ref[...], preferred_element_type=f32) * dinv


def _tc4_body(a0_ref, a1_ref, gf_ref, dinv_ref, bf_ref, out_ref):
    out_ref[...] = dinv_ref[...] * (a0_ref[0] + a1_ref[0] + gf_ref[...]) \
        + bf_ref[...]


def _row_spec(w):
    return pl.BlockSpec((R, w), lambda i: (i, 0))


def _const_spec(h, w):
    return pl.BlockSpec((h, w), lambda i: (0, 0))


def _half_spec(c, w):
    return pl.BlockSpec((1, R, w), lambda i, c=c: (c, i, 0))


_GRID = N // R

_tc1_call = pl.pallas_call(
    _tc1_body,
    grid=(_GRID,),
    in_specs=[_row_spec(C), _const_spec(C, C), _half_spec(0, 1),
              _half_spec(1, 1)],
    out_specs=[_row_spec(C), _row_spec(1)],
    out_shape=[jax.ShapeDtypeStruct((N, C), f32),
               jax.ShapeDtypeStruct((N, 1), f32)],
)


def _make_tc2(cout):
    return pl.pallas_call(
        _tc2_body,
        grid=(_GRID,),
        in_specs=[_half_spec(0, C), _half_spec(1, C), _row_spec(C),
                  _row_spec(1), _const_spec(1, C), _const_spec(C, cout)],
        out_specs=_row_spec(cout),
        out_shape=jax.ShapeDtypeStruct((N, cout), f32),
    )


_tc2_call = _make_tc2(C)
_tc3_call = _make_tc2(1)

_tc4_call = pl.pallas_call(
    _tc4_body,
    grid=(_GRID,),
    in_specs=[_half_spec(0, 1), _half_spec(1, 1), _row_spec(1), _row_spec(1),
              _const_spec(1, 1)],
    out_specs=_row_spec(1),
    out_shape=jax.ShapeDtypeStruct((N, 1), f32),
)


@jax.jit
def kernel(x, edge_index, batch, W0, b0, W1, b1, Wf, bf):
    src = edge_index[0].reshape(E // K, K)
    dst = edge_index[1].reshape(E // K, K)

    d = _deg_call(dst).reshape(2, NPAD, 1)
    g0, dinv = _tc1_call(x, W0, d, d)
    p = _edge_call(g0, src, dst)
    g1 = _tc2_call(p, p, g0, dinv, b0.reshape(1, C), W1)
    q = _edge_call(g1, src, dst)
    gf = _tc3_call(q, q, g1, dinv, b1.reshape(1, C), Wf)
    a = _fin_call(gf.reshape(N), src, dst).reshape(2, NPAD, 1)
    out = _tc4_call(a, a, gf, dinv, bf.reshape(1, 1))
    return out


# speedup vs baseline: 1.2622x; 1.2622x over previous
import jax
import jax.numpy as jnp
from jax import lax
from jax.experimental import pallas as pl
from jax.experimental.pallas import tpu as pltpu
from jax.experimental.pallas import tpu_sc as plsc

N = 10000
E = 320000
C = 128
NPAD = 10240
RPT = 640
K = 80
R = 1000
EPW = E // 32
NBLK = EPW // 80

_mesh = plsc.VectorSubcoreMesh(core_axis_name="c", subcore_axis_name="s")
f32 = jnp.float32


def _fill_vec(ref, n, val):
    def body(j, _):
        ref[pl.ds(j * 16, 16)] = jnp.full((16,), val, f32)
        return 0
    lax.fori_loop(0, n // 16, body, 0)


NR = NPAD // C


def _zero_2d(ref, rows):
    def body(r, _):
        for c4 in range(C // 16):
            ref[r, pl.ds(c4 * 16, 16)] = jnp.zeros((16,), f32)
        return 0
    lax.fori_loop(0, rows, body, 0)


def _fill_iota(ref, n):
    def body(j, _):
        ref[pl.ds(j * 16, 16)] = jnp.arange(16, dtype=jnp.int32) + j * 16
        return 0
    lax.fori_loop(0, n // 16, body, 0)


def _drain_and_writeback(acc_l, acc_s, idt, out_hbm, cid, sid, wbuf):
    pltpu.sync_copy(acc_l, acc_s.at[idt], add=True)
    plsc.subcore_barrier()
    rows = NR // 16
    pltpu.sync_copy(acc_s.at[pl.ds(sid * rows, rows)], wbuf)
    pltpu.sync_copy(wbuf, out_hbm.at[cid, pl.ds(sid * rows, rows)])


def _deg_body(dst_hbm, d_hbm, acc_s, acc_l, idt, dbig, wbuf):
    cid = lax.axis_index("c")
    sid = lax.axis_index("s")
    _zero_2d(acc_l, NR)
    _fill_iota(idt, NR)
    rows = NR // 16
    pltpu.sync_copy(acc_l.at[pl.ds(0, rows)], acc_s.at[pl.ds(sid * rows, rows)])
    wid = sid * 2 + cid
    pltpu.sync_copy(dst_hbm.at[pl.ds(wid * NBLK, NBLK)], dbig)
    plsc.subcore_barrier()

    ones16 = jnp.ones((16,), f32)

    def ebody(i, _):
        for j in range(K // 16):
            d16 = dbig[i, pl.ds(j * 16, 16)]
            row = lax.shift_right_logical(d16, 7)
            col = jnp.bitwise_and(d16, 127)
            plsc.addupdate_scatter(acc_l, [row, col], ones16)
        return 0
    lax.fori_loop(0, NBLK, ebody, 0)
    plsc.subcore_barrier()
    _drain_and_writeback(acc_l, acc_s, idt, d_hbm, cid, sid, wbuf)


_deg_call = pl.kernel(
    _deg_body,
    out_type=jax.ShapeDtypeStruct((2, NR, C), f32),
    mesh=_mesh,
    compiler_params=pltpu.CompilerParams(use_tc_tiling_on_sc=False, needs_layout_passes=False),
    scratch_types=[
        pltpu.VMEM_SHARED((NR, C), f32),
        pltpu.VMEM((NR, C), f32),
        pltpu.VMEM((NR,), jnp.int32),
        pltpu.VMEM((NBLK, K), jnp.int32),
        pltpu.VMEM((NR // 16, C), f32),
    ],
)


CH = 25


def _edge_body(g_hbm, src_hbm, dst_hbm, p_hbm, acc_s,
               buf0, buf1, sbig0, dbig0, sbig1, dbig1, sem0, sem1,
               isem0, isem1):
    cid = lax.axis_index("c")
    sid = lax.axis_index("s")

    def zrow(r, _):
        for c4 in range(C // 16):
            buf0[r, pl.ds(c4 * 16, 16)] = jnp.zeros((16,), f32)
        return 0
    lax.fori_loop(0, K, zrow, 0)

    def zcp(k, _):
        pltpu.sync_copy(buf0, acc_s.at[pl.ds(sid * RPT + k * K, K)])
        return 0
    lax.fori_loop(0, RPT // K, zcp, 0)
    plsc.subcore_barrier()

    wid = sid * 2 + cid

    def stage(c, sb, db, isem):
        rowbase = wid * NBLK + c * CH
        pltpu.async_copy(src_hbm.at[pl.ds(rowbase, CH)], sb, isem)
        pltpu.async_copy(dst_hbm.at[pl.ds(rowbase, CH)], db, isem)

    def stage_wait(c, sb, db, isem):
        rowbase = wid * NBLK + c * CH
        pltpu.make_async_copy(src_hbm.at[pl.ds(rowbase, CH)], sb, isem).wait()
        pltpu.make_async_copy(dst_hbm.at[pl.ds(rowbase, CH)], db, isem).wait()

    def process(sb, db):
        def start(b, bf, sem):
            pltpu.async_copy(g_hbm.at[sb.at[b]], bf, sem)

        def drain(b, bf, sem):
            pltpu.make_async_copy(g_hbm.at[sb.at[b]], bf, sem).wait()
            pltpu.sync_copy(bf, acc_s.at[db.at[b]], add=True)

        start(0, buf0, sem0)

        def pair(o, _):
            start(2 * o + 1, buf1, sem1)
            drain(2 * o, buf0, sem0)
            start(2 * o + 2, buf0, sem0)
            drain(2 * o + 1, buf1, sem1)
            return 0
        lax.fori_loop(0, CH // 2, pair, 0)
        drain(CH - 1, buf0, sem0)

    stage(0, sbig0, dbig0, isem0)
    stage_wait(0, sbig0, dbig0, isem0)

    def two(o, _):
        stage(2 * o + 1, sbig1, dbig1, isem1)
        process(sbig0, dbig0)
        stage_wait(2 * o + 1, sbig1, dbig1, isem1)
        stage(2 * o + 2, sbig0, dbig0, isem0)
        process(sbig1, dbig1)
        stage_wait(2 * o + 2, sbig0, dbig0, isem0)
        return 0
    lax.fori_loop(0, (NBLK // CH) // 2, two, 0)
    process(sbig0, dbig0)
    plsc.subcore_barrier()

    def wb(k, _):
        base = sid * RPT + k * K
        pltpu.sync_copy(acc_s.at[pl.ds(base, K)], buf0)
        pltpu.sync_copy(buf0, p_hbm.at[cid, pl.ds(base, K)])
        return 0
    lax.fori_loop(0, RPT // K, wb, 0)


_edge_call = pl.kernel(
    _edge_body,
    out_type=jax.ShapeDtypeStruct((2, NPAD, C), f32),
    mesh=_mesh,
    compiler_params=pltpu.CompilerParams(use_tc_tiling_on_sc=False, needs_layout_passes=False),
    scratch_types=[
        pltpu.VMEM_SHARED((NPAD, C), f32),
        pltpu.VMEM((K, C), f32),
        pltpu.VMEM((K, C), f32),
        pltpu.VMEM((CH, K), jnp.int32),
        pltpu.VMEM((CH, K), jnp.int32),
        pltpu.VMEM((CH, K), jnp.int32),
        pltpu.VMEM((CH, K), jnp.int32),
        pltpu.SemaphoreType.DMA,
        pltpu.SemaphoreType.DMA,
        pltpu.SemaphoreType.DMA,
        pltpu.SemaphoreType.DMA,
    ],
)


NBLK2 = (E // K) // 16


def _fin_body(gf_hbm, src_hbm, dst_hbm, dinv_hbm, bf_hbm, o_hbm,
              acc_s, acc_l, gf_v, idt, sbig, dbig, cbuf, dv, obuf, bfv):
    cid = lax.axis_index("c")
    sid = lax.axis_index("s")
    _zero_2d(acc_l, NR)
    _fill_iota(idt, NR)
    rows = NR // 16
    pltpu.sync_copy(acc_l.at[pl.ds(0, rows)], acc_s.at[pl.ds(sid * rows, rows)])
    pltpu.sync_copy(gf_hbm, gf_v.at[pl.ds(0, N)])
    pltpu.sync_copy(src_hbm.at[pl.ds(sid * NBLK2, NBLK2)], sbig)
    pltpu.sync_copy(dst_hbm.at[pl.ds(sid * NBLK2, NBLK2)], dbig)
    plsc.subcore_barrier()

    def ebody(i, _):
        for j in range(K // 16):
            s16 = sbig[i, pl.ds(j * 16, 16)]
            d16 = dbig[i, pl.ds(j * 16, 16)]
            vals = plsc.load_gather(gf_v, [s16])
            row = lax.shift_right_logical(d16, 7)
            col = jnp.bitwise_and(d16, 127)
            plsc.addupdate_scatter(acc_l, [row, col], vals)
        return 0
    lax.fori_loop(0, NBLK2, ebody, 0)
    plsc.subcore_barrier()
    pltpu.sync_copy(acc_l, acc_s.at[idt], add=True)
    plsc.subcore_barrier()

    @pl.when(cid == 0)
    def _():
        pltpu.sync_copy(acc_s.at[pl.ds(sid * (NR // 16), NR // 16)], cbuf)
        pltpu.sync_copy(dinv_hbm.at[pl.ds(sid * RPT, RPT)], dv)
        pltpu.sync_copy(bf_hbm, bfv)
        b16 = bfv[...]

        def comb(j, _):
            row = lax.shift_right_logical(j, 3)
            col = jnp.bitwise_and(j, 7) * 16
            a16 = cbuf[row, pl.ds(col, 16)]
            g16 = gf_v[pl.ds(sid * RPT + j * 16, 16)]
            d16 = dv[pl.ds(j * 16, 16)]
            obuf[pl.ds(j * 16, 16)] = d16 * (a16 + g16) + b16
            return 0
        lax.fori_loop(0, RPT // 16, comb, 0)
        pltpu.sync_copy(obuf, o_hbm.at[pl.ds(sid * RPT, RPT)])


_fin_call = pl.kernel(
    _fin_body,
    out_type=jax.ShapeDtypeStruct((NPAD,), f32),
    mesh=_mesh,
    compiler_params=pltpu.CompilerParams(use_tc_tiling_on_sc=False, needs_layout_passes=False),
    scratch_types=[
        pltpu.VMEM_SHARED((NR, C), f32),
        pltpu.VMEM((NR, C), f32),
        pltpu.VMEM((NPAD,), f32),
        pltpu.VMEM((NR,), jnp.int32),
        pltpu.VMEM((NBLK2, K), jnp.int32),
        pltpu.VMEM((NBLK2, K), jnp.int32),
        pltpu.VMEM((NR // 16, C), f32),
        pltpu.VMEM((RPT,), f32),
        pltpu.VMEM((RPT,), f32),
        pltpu.VMEM((16,), f32),
    ],
)


def _tc1_body(x_ref, w_ref, d0_ref, d1_ref, g_ref, dinv_ref):
    dinv = lax.rsqrt(d0_ref[0] + d1_ref[0] + 1.0)
    g_ref[...] = jnp.dot(x_ref[...], w_ref[...],
                         preferred_element_type=f32) * dinv
    dinv_ref[...] = dinv


def _tc2_body(p0_ref, p1_ref, g_ref, dinv_ref, b_ref, w_ref, o_ref):
    dinv = dinv_ref[...]
    h = jnp.maximum(
        dinv * (p0_ref[0] + p1_ref[0] + g_ref[...]) + b_ref[...], 0.0)
    o_ref[...] = jnp.dot(h, w_ref[...], preferred_element_type=f32) * dinv


def _row_spec(w):
    return pl.BlockSpec((R, w), lambda i: (i, 0))


def _const_spec(h, w):
    return pl.BlockSpec((h, w), lambda i: (0, 0))


def _half_spec(c, w):
    return pl.BlockSpec((1, R, w), lambda i, c=c: (c, i, 0))


_GRID = N // R

_tc1_call = pl.pallas_call(
    _tc1_body,
    grid=(_GRID,),
    in_specs=[_row_spec(C), _const_spec(C, C), _half_spec(0, 1),
              _half_spec(1, 1)],
    out_specs=[_row_spec(C), _row_spec(1)],
    out_shape=[jax.ShapeDtypeStruct((N, C), f32),
               jax.ShapeDtypeStruct((NPAD, 1), f32)],
)


def _make_tc2(cout):
    return pl.pallas_call(
        _tc2_body,
        grid=(_GRID,),
        in_specs=[_half_spec(0, C), _half_spec(1, C), _row_spec(C),
                  _row_spec(1), _const_spec(1, C), _const_spec(C, cout)],
        out_specs=_row_spec(cout),
        out_shape=jax.ShapeDtypeStruct((N, cout), f32),
    )


_tc2_call = _make_tc2(C)
_tc3_call = _make_tc2(1)

@jax.jit
def kernel(x, edge_index, batch, W0, b0, W1, b1, Wf, bf):
    src = edge_index[0].reshape(E // K, K)
    dst = edge_index[1].reshape(E // K, K)

    d = _deg_call(dst).reshape(2, NPAD, 1)
    g0, dinv = _tc1_call(x, W0, d, d)
    p = _edge_call(g0, src, dst)
    g1 = _tc2_call(p, p, g0, dinv, b0.reshape(1, C), W1)
    q = _edge_call(g1, src, dst)
    gf = _tc3_call(q, q, g1, dinv, b1.reshape(1, C), Wf)
    outp = _fin_call(gf.reshape(N), src, dst, dinv.reshape(NPAD),
                     jnp.broadcast_to(bf, (16,)))
    return outp[:N].reshape(N, 1)


# speedup vs baseline: 1.2891x; 1.0214x over previous
import jax
import jax.numpy as jnp
from jax import lax
from jax.experimental import pallas as pl
from jax.experimental.pallas import tpu as pltpu
from jax.experimental.pallas import tpu_sc as plsc

N = 10000
E = 320000
C = 128
NPAD = 10240
RPT = 640
K = 80
R = 1000
EPW = E // 32
NBLK = EPW // 80

_mesh = plsc.VectorSubcoreMesh(core_axis_name="c", subcore_axis_name="s")
f32 = jnp.float32


def _fill_vec(ref, n, val):
    def body(j, _):
        ref[pl.ds(j * 16, 16)] = jnp.full((16,), val, f32)
        return 0
    lax.fori_loop(0, n // 16, body, 0)


NR = NPAD // C


def _zero_2d(ref, rows):
    def body(r, _):
        for c4 in range(C // 16):
            ref[r, pl.ds(c4 * 16, 16)] = jnp.zeros((16,), f32)
        return 0
    lax.fori_loop(0, rows, body, 0)


def _fill_iota(ref, n):
    def body(j, _):
        ref[pl.ds(j * 16, 16)] = jnp.arange(16, dtype=jnp.int32) + j * 16
        return 0
    lax.fori_loop(0, n // 16, body, 0)


def _drain_and_writeback(acc_l, acc_s, idt, out_hbm, cid, sid, wbuf):
    pltpu.sync_copy(acc_l, acc_s.at[idt], add=True)
    plsc.subcore_barrier()
    rows = NR // 16
    pltpu.sync_copy(acc_s.at[pl.ds(sid * rows, rows)], wbuf)
    pltpu.sync_copy(wbuf, out_hbm.at[cid, pl.ds(sid * rows, rows)])


def _deg_body(dst_hbm, d_hbm, acc_s, acc_l, idt, dbig, wbuf):
    cid = lax.axis_index("c")
    sid = lax.axis_index("s")
    _zero_2d(acc_l, NR)
    _fill_iota(idt, NR)
    rows = NR // 16
    pltpu.sync_copy(acc_l.at[pl.ds(0, rows)], acc_s.at[pl.ds(sid * rows, rows)])
    wid = sid * 2 + cid
    pltpu.sync_copy(dst_hbm.at[pl.ds(wid * NBLK, NBLK)], dbig)
    plsc.subcore_barrier()

    ones16 = jnp.ones((16,), f32)

    def ebody(i, _):
        for j in range(K // 16):
            d16 = dbig[i, pl.ds(j * 16, 16)]
            row = lax.shift_right_logical(d16, 7)
            col = jnp.bitwise_and(d16, 127)
            plsc.addupdate_scatter(acc_l, [row, col], ones16)
        return 0
    lax.fori_loop(0, NBLK, ebody, 0)
    plsc.subcore_barrier()
    _drain_and_writeback(acc_l, acc_s, idt, d_hbm, cid, sid, wbuf)


_deg_call = pl.kernel(
    _deg_body,
    out_type=jax.ShapeDtypeStruct((2, NR, C), f32),
    mesh=_mesh,
    compiler_params=pltpu.CompilerParams(use_tc_tiling_on_sc=False, needs_layout_passes=False),
    scratch_types=[
        pltpu.VMEM_SHARED((NR, C), f32),
        pltpu.VMEM((NR, C), f32),
        pltpu.VMEM((NR,), jnp.int32),
        pltpu.VMEM((NBLK, K), jnp.int32),
        pltpu.VMEM((NR // 16, C), f32),
    ],
)


def _edge_body(g_hbm, src_hbm, dst_hbm, p_hbm, acc_s,
               buf0, buf1, sbig, dbig, sem0, sem1):
    cid = lax.axis_index("c")
    sid = lax.axis_index("s")

    def zrow(r, _):
        for c4 in range(C // 16):
            buf0[r, pl.ds(c4 * 16, 16)] = jnp.zeros((16,), f32)
        return 0
    lax.fori_loop(0, K, zrow, 0)

    def zcp(k, _):
        pltpu.sync_copy(buf0, acc_s.at[pl.ds(sid * RPT + k * K, K)])
        return 0
    lax.fori_loop(0, RPT // K, zcp, 0)
    plsc.subcore_barrier()

    wid = sid * 2 + cid

    pltpu.sync_copy(src_hbm.at[pl.ds(wid * NBLK, NBLK)], sbig)
    pltpu.sync_copy(dst_hbm.at[pl.ds(wid * NBLK, NBLK)], dbig)

    def start(b, bf, sem):
        pltpu.async_copy(g_hbm.at[sbig.at[b]], bf, sem)

    def drain(b, bf, sem):
        pltpu.make_async_copy(g_hbm.at[sbig.at[b]], bf, sem).wait()
        pltpu.sync_copy(bf, acc_s.at[dbig.at[b]], add=True)

    start(0, buf0, sem0)

    def pair(o, _):
        start(2 * o + 1, buf1, sem1)
        drain(2 * o, buf0, sem0)
        start(2 * o + 2, buf0, sem0)
        drain(2 * o + 1, buf1, sem1)
        return 0
    lax.fori_loop(0, NBLK // 2, pair, 0)
    drain(NBLK - 1, buf0, sem0)
    plsc.subcore_barrier()

    def wb(k, _):
        base = sid * RPT + k * K
        pltpu.sync_copy(acc_s.at[pl.ds(base, K)], buf0)
        pltpu.sync_copy(buf0, p_hbm.at[cid, pl.ds(base, K)])
        return 0
    lax.fori_loop(0, RPT // K, wb, 0)


_edge_call = pl.kernel(
    _edge_body,
    out_type=jax.ShapeDtypeStruct((2, NPAD, C), f32),
    mesh=_mesh,
    compiler_params=pltpu.CompilerParams(use_tc_tiling_on_sc=False, needs_layout_passes=False),
    scratch_types=[
        pltpu.VMEM_SHARED((NPAD, C), f32),
        pltpu.VMEM((K, C), f32),
        pltpu.VMEM((K, C), f32),
        pltpu.VMEM((NBLK, K), jnp.int32),
        pltpu.VMEM((NBLK, K), jnp.int32),
        pltpu.SemaphoreType.DMA,
        pltpu.SemaphoreType.DMA,
    ],
)


NBLK2 = (E // K) // 16


def _fin_body(gf_hbm, src_hbm, dst_hbm, dinv_hbm, bf_hbm, o_hbm,
              acc_s, acc_l, gf_v, idt, sbig, dbig, cbuf, dv, obuf, bfv):
    cid = lax.axis_index("c")
    sid = lax.axis_index("s")
    _zero_2d(acc_l, NR)
    _fill_iota(idt, NR)
    rows = NR // 16
    pltpu.sync_copy(acc_l.at[pl.ds(0, rows)], acc_s.at[pl.ds(sid * rows, rows)])
    pltpu.sync_copy(gf_hbm, gf_v.at[pl.ds(0, N)])
    pltpu.sync_copy(src_hbm.at[pl.ds(sid * NBLK2, NBLK2)], sbig)
    pltpu.sync_copy(dst_hbm.at[pl.ds(sid * NBLK2, NBLK2)], dbig)
    plsc.subcore_barrier()

    def ebody(i, _):
        for j in range(K // 16):
            s16 = sbig[i, pl.ds(j * 16, 16)]
            d16 = dbig[i, pl.ds(j * 16, 16)]
            vals = plsc.load_gather(gf_v, [s16])
            row = lax.shift_right_logical(d16, 7)
            col = jnp.bitwise_and(d16, 127)
            plsc.addupdate_scatter(acc_l, [row, col], vals)
        return 0
    lax.fori_loop(0, NBLK2, ebody, 0)
    plsc.subcore_barrier()
    pltpu.sync_copy(acc_l, acc_s.at[idt], add=True)
    plsc.subcore_barrier()

    @pl.when(cid == 0)
    def _():
        pltpu.sync_copy(acc_s.at[pl.ds(sid * (NR // 16), NR // 16)], cbuf)
        pltpu.sync_copy(dinv_hbm.at[pl.ds(sid * RPT, RPT)], dv)
        pltpu.sync_copy(bf_hbm, bfv)
        b16 = bfv[...]

        def comb(j, _):
            row = lax.shift_right_logical(j, 3)
            col = jnp.bitwise_and(j, 7) * 16
            a16 = cbuf[row, pl.ds(col, 16)]
            g16 = gf_v[pl.ds(sid * RPT + j * 16, 16)]
            d16 = dv[pl.ds(j * 16, 16)]
            obuf[pl.ds(j * 16, 16)] = d16 * (a16 + g16) + b16
            return 0
        lax.fori_loop(0, RPT // 16, comb, 0)
        pltpu.sync_copy(obuf, o_hbm.at[pl.ds(sid * RPT, RPT)])


_fin_call = pl.kernel(
    _fin_body,
    out_type=jax.ShapeDtypeStruct((NPAD,), f32),
    mesh=_mesh,
    compiler_params=pltpu.CompilerParams(use_tc_tiling_on_sc=False, needs_layout_passes=False),
    scratch_types=[
        pltpu.VMEM_SHARED((NR, C), f32),
        pltpu.VMEM((NR, C), f32),
        pltpu.VMEM((NPAD,), f32),
        pltpu.VMEM((NR,), jnp.int32),
        pltpu.VMEM((NBLK2, K), jnp.int32),
        pltpu.VMEM((NBLK2, K), jnp.int32),
        pltpu.VMEM((NR // 16, C), f32),
        pltpu.VMEM((RPT,), f32),
        pltpu.VMEM((RPT,), f32),
        pltpu.VMEM((16,), f32),
    ],
)


def _tc1_body(x_ref, w_ref, d0_ref, d1_ref, g_ref, dinv_ref):
    dinv = lax.rsqrt(d0_ref[0] + d1_ref[0] + 1.0)
    g_ref[...] = jnp.dot(x_ref[...], w_ref[...],
                         preferred_element_type=f32) * dinv
    dinv_ref[...] = dinv


def _tc2_body(p0_ref, p1_ref, g_ref, dinv_ref, b_ref, w_ref, o_ref):
    dinv = dinv_ref[...]
    h = jnp.maximum(
        dinv * (p0_ref[0] + p1_ref[0] + g_ref[...]) + b_ref[...], 0.0)
    o_ref[...] = jnp.dot(h, w_ref[...], preferred_element_type=f32) * dinv


def _row_spec(w):
    return pl.BlockSpec((R, w), lambda i: (i, 0))


def _const_spec(h, w):
    return pl.BlockSpec((h, w), lambda i: (0, 0))


def _half_spec(c, w):
    return pl.BlockSpec((1, R, w), lambda i, c=c: (c, i, 0))


_GRID = N // R

_tc1_call = pl.pallas_call(
    _tc1_body,
    grid=(_GRID,),
    in_specs=[_row_spec(C), _const_spec(C, C), _half_spec(0, 1),
              _half_spec(1, 1)],
    out_specs=[_row_spec(C), _row_spec(1)],
    out_shape=[jax.ShapeDtypeStruct((N, C), f32),
               jax.ShapeDtypeStruct((NPAD, 1), f32)],
)


def _make_tc2(cout):
    return pl.pallas_call(
        _tc2_body,
        grid=(_GRID,),
        in_specs=[_half_spec(0, C), _half_spec(1, C), _row_spec(C),
                  _row_spec(1), _const_spec(1, C), _const_spec(C, cout)],
        out_specs=_row_spec(cout),
        out_shape=jax.ShapeDtypeStruct((N, cout), f32),
    )


_tc2_call = _make_tc2(C)
_tc3_call = _make_tc2(1)

@jax.jit
def kernel(x, edge_index, batch, W0, b0, W1, b1, Wf, bf):
    src = edge_index[0].reshape(E // K, K)
    dst = edge_index[1].reshape(E // K, K)

    d = _deg_call(dst).reshape(2, NPAD, 1)
    g0, dinv = _tc1_call(x, W0, d, d)
    p = _edge_call(g0, src, dst)
    g1 = _tc2_call(p, p, g0, dinv, b0.reshape(1, C), W1)
    q = _edge_call(g1, src, dst)
    gf = _tc3_call(q, q, g1, dinv, b1.reshape(1, C), Wf)
    outp = _fin_call(gf.reshape(N), src, dst, dinv.reshape(NPAD),
                     jnp.broadcast_to(bf, (16,)))
    return outp[:N].reshape(N, 1)


# speedup vs baseline: 1.3212x; 1.0249x over previous
import jax
import jax.numpy as jnp
from jax import lax
from jax.experimental import pallas as pl
from jax.experimental.pallas import tpu as pltpu
from jax.experimental.pallas import tpu_sc as plsc

N = 10000
E = 320000
C = 128
NPAD = 10240
RPT = 640
K = 80
R = 1000
EPW = E // 32
NBLK = EPW // 80

_mesh = plsc.VectorSubcoreMesh(core_axis_name="c", subcore_axis_name="s")
f32 = jnp.float32


def _fill_vec(ref, n, val):
    def body(j, _):
        ref[pl.ds(j * 16, 16)] = jnp.full((16,), val, f32)
        return 0
    lax.fori_loop(0, n // 16, body, 0)


NR = NPAD // C


def _zero_2d(ref, rows):
    def body(r, _):
        for c4 in range(C // 16):
            ref[r, pl.ds(c4 * 16, 16)] = jnp.zeros((16,), f32)
        return 0
    lax.fori_loop(0, rows, body, 0)


def _fill_iota(ref, n):
    def body(j, _):
        ref[pl.ds(j * 16, 16)] = jnp.arange(16, dtype=jnp.int32) + j * 16
        return 0
    lax.fori_loop(0, n // 16, body, 0)


def _drain_and_writeback(acc_l, acc_s, idt, out_hbm, cid, sid, wbuf):
    pltpu.sync_copy(acc_l, acc_s.at[idt], add=True)
    plsc.subcore_barrier()
    rows = NR // 16
    pltpu.sync_copy(acc_s.at[pl.ds(sid * rows, rows)], wbuf)
    pltpu.sync_copy(wbuf, out_hbm.at[cid, pl.ds(sid * rows, rows)])


def _deg_body(dst_hbm, d_hbm, acc_s, acc_l, idt, dbig, wbuf):
    cid = lax.axis_index("c")
    sid = lax.axis_index("s")
    _zero_2d(acc_l, NR)
    _fill_iota(idt, NR)
    rows = NR // 16
    pltpu.sync_copy(acc_l.at[pl.ds(0, rows)], acc_s.at[pl.ds(sid * rows, rows)])
    wid = sid * 2 + cid
    pltpu.sync_copy(dst_hbm.at[pl.ds(wid * NBLK, NBLK)], dbig)
    plsc.subcore_barrier()

    ones16 = jnp.ones((16,), f32)

    def ebody(i, _):
        for j in range(K // 16):
            d16 = dbig[i, pl.ds(j * 16, 16)]
            row = lax.shift_right_logical(d16, 7)
            col = jnp.bitwise_and(d16, 127)
            plsc.addupdate_scatter(acc_l, [row, col], ones16)
        return 0
    lax.fori_loop(0, NBLK, ebody, 0)
    plsc.subcore_barrier()
    _drain_and_writeback(acc_l, acc_s, idt, d_hbm, cid, sid, wbuf)


_deg_call = pl.kernel(
    _deg_body,
    out_type=jax.ShapeDtypeStruct((2, NR, C), f32),
    mesh=_mesh,
    compiler_params=pltpu.CompilerParams(use_tc_tiling_on_sc=False, needs_layout_passes=False),
    scratch_types=[
        pltpu.VMEM_SHARED((NR, C), f32),
        pltpu.VMEM((NR, C), f32),
        pltpu.VMEM((NR,), jnp.int32),
        pltpu.VMEM((NBLK, K), jnp.int32),
        pltpu.VMEM((NR // 16, C), f32),
    ],
)


def _edge_body(g_hbm, src_hbm, dst_hbm, p_hbm, acc_s,
               buf0, buf1, sbig, dbig, sem0, sem1):
    cid = lax.axis_index("c")
    sid = lax.axis_index("s")

    wid = sid * 2 + cid

    pltpu.async_copy(src_hbm.at[pl.ds(wid * NBLK, NBLK)], sbig, sem0)
    pltpu.async_copy(dst_hbm.at[pl.ds(wid * NBLK, NBLK)], dbig, sem1)

    def zrow(r, _):
        for c4 in range(C // 16):
            buf0[r, pl.ds(c4 * 16, 16)] = jnp.zeros((16,), f32)
        return 0
    lax.fori_loop(0, K, zrow, 0)

    def zcp(k, _):
        pltpu.sync_copy(buf0, acc_s.at[pl.ds(sid * RPT + k * K, K)])
        return 0
    lax.fori_loop(0, RPT // K, zcp, 0)
    pltpu.make_async_copy(src_hbm.at[pl.ds(wid * NBLK, NBLK)], sbig, sem0).wait()
    pltpu.make_async_copy(dst_hbm.at[pl.ds(wid * NBLK, NBLK)], dbig, sem1).wait()
    plsc.subcore_barrier()

    def start(b, bf, sem):
        pltpu.async_copy(g_hbm.at[sbig.at[b]], bf, sem)

    def drain(b, bf, sem):
        pltpu.make_async_copy(g_hbm.at[sbig.at[b]], bf, sem).wait()
        pltpu.sync_copy(bf, acc_s.at[dbig.at[b]], add=True)

    start(0, buf0, sem0)

    def pair(o, _):
        start(2 * o + 1, buf1, sem1)
        drain(2 * o, buf0, sem0)
        start(2 * o + 2, buf0, sem0)
        drain(2 * o + 1, buf1, sem1)
        return 0
    lax.fori_loop(0, NBLK // 2, pair, 0)
    drain(NBLK - 1, buf0, sem0)
    plsc.subcore_barrier()

    def s2v(k, bf, sem):
        pltpu.async_copy(acc_s.at[pl.ds(sid * RPT + k * K, K)], bf, sem)

    def s2v_wait(k, bf, sem):
        pltpu.make_async_copy(
            acc_s.at[pl.ds(sid * RPT + k * K, K)], bf, sem).wait()

    def v2h(k, bf, sem):
        pltpu.async_copy(bf, p_hbm.at[cid, pl.ds(sid * RPT + k * K, K)], sem)

    def v2h_wait(k, bf, sem):
        pltpu.make_async_copy(
            bf, p_hbm.at[cid, pl.ds(sid * RPT + k * K, K)], sem).wait()

    nwb = RPT // K
    bufs = [(buf0, sem0), (buf1, sem1)]
    s2v(0, *bufs[0])
    for k in range(nwb):
        cur = bufs[k % 2]
        oth = bufs[(k + 1) % 2]
        s2v_wait(k, *cur)
        if k >= 1:
            v2h_wait(k - 1, *oth)
        if k < nwb - 1:
            s2v(k + 1, *oth)
        v2h(k, *cur)
    v2h_wait(nwb - 1, *bufs[(nwb - 1) % 2])


_edge_call = pl.kernel(
    _edge_body,
    out_type=jax.ShapeDtypeStruct((2, NPAD, C), f32),
    mesh=_mesh,
    compiler_params=pltpu.CompilerParams(use_tc_tiling_on_sc=False, needs_layout_passes=False),
    scratch_types=[
        pltpu.VMEM_SHARED((NPAD, C), f32),
        pltpu.VMEM((K, C), f32),
        pltpu.VMEM((K, C), f32),
        pltpu.VMEM((NBLK, K), jnp.int32),
        pltpu.VMEM((NBLK, K), jnp.int32),
        pltpu.SemaphoreType.DMA,
        pltpu.SemaphoreType.DMA,
    ],
)


NBLK2 = (E // K) // 16


def _fin_body(gf_hbm, src_hbm, dst_hbm, dinv_hbm, bf_hbm, o_hbm,
              acc_s, acc_l, gf_v, idt, sbig, dbig, cbuf, dv, obuf, bfv):
    cid = lax.axis_index("c")
    sid = lax.axis_index("s")
    _zero_2d(acc_l, NR)
    _fill_iota(idt, NR)
    rows = NR // 16
    pltpu.sync_copy(acc_l.at[pl.ds(0, rows)], acc_s.at[pl.ds(sid * rows, rows)])
    pltpu.sync_copy(gf_hbm, gf_v.at[pl.ds(0, N)])
    pltpu.sync_copy(src_hbm.at[pl.ds(sid * NBLK2, NBLK2)], sbig)
    pltpu.sync_copy(dst_hbm.at[pl.ds(sid * NBLK2, NBLK2)], dbig)
    plsc.subcore_barrier()

    def ebody(i, _):
        for j in range(K // 16):
            s16 = sbig[i, pl.ds(j * 16, 16)]
            d16 = dbig[i, pl.ds(j * 16, 16)]
            vals = plsc.load_gather(gf_v, [s16])
            row = lax.shift_right_logical(d16, 7)
            col = jnp.bitwise_and(d16, 127)
            plsc.addupdate_scatter(acc_l, [row, col], vals)
        return 0
    lax.fori_loop(0, NBLK2, ebody, 0)
    plsc.subcore_barrier()
    pltpu.sync_copy(acc_l, acc_s.at[idt], add=True)
    plsc.subcore_barrier()

    @pl.when(cid == 0)
    def _():
        pltpu.sync_copy(acc_s.at[pl.ds(sid * (NR // 16), NR // 16)], cbuf)
        pltpu.sync_copy(dinv_hbm.at[pl.ds(sid * RPT, RPT)], dv)
        pltpu.sync_copy(bf_hbm, bfv)
        b16 = bfv[...]

        def comb(j, _):
            row = lax.shift_right_logical(j, 3)
            col = jnp.bitwise_and(j, 7) * 16
            a16 = cbuf[row, pl.ds(col, 16)]
            g16 = gf_v[pl.ds(sid * RPT + j * 16, 16)]
            d16 = dv[pl.ds(j * 16, 16)]
            obuf[pl.ds(j * 16, 16)] = d16 * (a16 + g16) + b16
            return 0
        lax.fori_loop(0, RPT // 16, comb, 0)
        pltpu.sync_copy(obuf, o_hbm.at[pl.ds(sid * RPT, RPT)])


_fin_call = pl.kernel(
    _fin_body,
    out_type=jax.ShapeDtypeStruct((NPAD,), f32),
    mesh=_mesh,
    compiler_params=pltpu.CompilerParams(use_tc_tiling_on_sc=False, needs_layout_passes=False),
    scratch_types=[
        pltpu.VMEM_SHARED((NR, C), f32),
        pltpu.VMEM((NR, C), f32),
        pltpu.VMEM((NPAD,), f32),
        pltpu.VMEM((NR,), jnp.int32),
        pltpu.VMEM((NBLK2, K), jnp.int32),
        pltpu.VMEM((NBLK2, K), jnp.int32),
        pltpu.VMEM((NR // 16, C), f32),
        pltpu.VMEM((RPT,), f32),
        pltpu.VMEM((RPT,), f32),
        pltpu.VMEM((16,), f32),
    ],
)


def _tc1_body(x_ref, w_ref, d0_ref, d1_ref, g_ref, dinv_ref):
    dinv = lax.rsqrt(d0_ref[0] + d1_ref[0] + 1.0)
    g_ref[...] = jnp.dot(x_ref[...], w_ref[...],
                         preferred_element_type=f32) * dinv
    dinv_ref[...] = dinv


def _tc2_body(p0_ref, p1_ref, g_ref, dinv_ref, b_ref, w_ref, o_ref):
    dinv = dinv_ref[...]
    h = jnp.maximum(
        dinv * (p0_ref[0] + p1_ref[0] + g_ref[...]) + b_ref[...], 0.0)
    o_ref[...] = jnp.dot(h, w_ref[...], preferred_element_type=f32) * dinv


def _row_spec(w):
    return pl.BlockSpec((R, w), lambda i: (i, 0))


def _const_spec(h, w):
    return pl.BlockSpec((h, w), lambda i: (0, 0))


def _half_spec(c, w):
    return pl.BlockSpec((1, R, w), lambda i, c=c: (c, i, 0))


_GRID = N // R

_tc1_call = pl.pallas_call(
    _tc1_body,
    grid=(_GRID,),
    in_specs=[_row_spec(C), _const_spec(C, C), _half_spec(0, 1),
              _half_spec(1, 1)],
    out_specs=[_row_spec(C), _row_spec(1)],
    out_shape=[jax.ShapeDtypeStruct((N, C), f32),
               jax.ShapeDtypeStruct((NPAD, 1), f32)],
)


def _make_tc2(cout):
    return pl.pallas_call(
        _tc2_body,
        grid=(_GRID,),
        in_specs=[_half_spec(0, C), _half_spec(1, C), _row_spec(C),
                  _row_spec(1), _const_spec(1, C), _const_spec(C, cout)],
        out_specs=_row_spec(cout),
        out_shape=jax.ShapeDtypeStruct((N, cout), f32),
    )


_tc2_call = _make_tc2(C)
_tc3_call = _make_tc2(1)

@jax.jit
def kernel(x, edge_index, batch, W0, b0, W1, b1, Wf, bf):
    src = edge_index[0].reshape(E // K, K)
    dst = edge_index[1].reshape(E // K, K)

    d = _deg_call(dst).reshape(2, NPAD, 1)
    g0, dinv = _tc1_call(x, W0, d, d)
    p = _edge_call(g0, src, dst)
    g1 = _tc2_call(p, p, g0, dinv, b0.reshape(1, C), W1)
    q = _edge_call(g1, src, dst)
    gf = _tc3_call(q, q, g1, dinv, b1.reshape(1, C), Wf)
    outp = _fin_call(gf.reshape(N), src, dst, dinv.reshape(NPAD),
                     jnp.broadcast_to(bf, (16,)))
    return outp[:N].reshape(N, 1)


# speedup vs baseline: 1.3499x; 1.0217x over previous
import jax
import jax.numpy as jnp
from jax import lax
from jax.experimental import pallas as pl
from jax.experimental.pallas import tpu as pltpu
from jax.experimental.pallas import tpu_sc as plsc

N = 10000
E = 320000
C = 128
NPAD = 10240
RPT = 640
K = 80
R = 2000
EPW = E // 32
NBLK = EPW // 80

_mesh = plsc.VectorSubcoreMesh(core_axis_name="c", subcore_axis_name="s")
f32 = jnp.float32


def _fill_vec(ref, n, val):
    def body(j, _):
        ref[pl.ds(j * 16, 16)] = jnp.full((16,), val, f32)
        return 0
    lax.fori_loop(0, n // 16, body, 0)


NR = NPAD // C


def _zero_2d(ref, rows):
    def body(r, _):
        for c4 in range(C // 16):
            ref[r, pl.ds(c4 * 16, 16)] = jnp.zeros((16,), f32)
        return 0
    lax.fori_loop(0, rows, body, 0)


def _fill_iota(ref, n):
    def body(j, _):
        ref[pl.ds(j * 16, 16)] = jnp.arange(16, dtype=jnp.int32) + j * 16
        return 0
    lax.fori_loop(0, n // 16, body, 0)


def _drain_and_writeback(acc_l, acc_s, idt, out_hbm, cid, sid, wbuf):
    pltpu.sync_copy(acc_l, acc_s.at[idt], add=True)
    plsc.subcore_barrier()
    rows = NR // 16
    pltpu.sync_copy(acc_s.at[pl.ds(sid * rows, rows)], wbuf)
    pltpu.sync_copy(wbuf, out_hbm.at[cid, pl.ds(sid * rows, rows)])


def _deg_body(dst_hbm, d_hbm, acc_s, acc_l, idt, dbig, wbuf):
    cid = lax.axis_index("c")
    sid = lax.axis_index("s")
    _zero_2d(acc_l, NR)
    _fill_iota(idt, NR)
    rows = NR // 16
    pltpu.sync_copy(acc_l.at[pl.ds(0, rows)], acc_s.at[pl.ds(sid * rows, rows)])
    wid = sid * 2 + cid
    pltpu.sync_copy(dst_hbm.at[pl.ds(wid * NBLK, NBLK)], dbig)
    plsc.subcore_barrier()

    ones16 = jnp.ones((16,), f32)

    def ebody(i, _):
        for j in range(K // 16):
            d16 = dbig[i, pl.ds(j * 16, 16)]
            row = lax.shift_right_logical(d16, 7)
            col = jnp.bitwise_and(d16, 127)
            plsc.addupdate_scatter(acc_l, [row, col], ones16)
        return 0
    lax.fori_loop(0, NBLK, ebody, 0)
    plsc.subcore_barrier()
    _drain_and_writeback(acc_l, acc_s, idt, d_hbm, cid, sid, wbuf)


_deg_call = pl.kernel(
    _deg_body,
    out_type=jax.ShapeDtypeStruct((2, NR, C), f32),
    mesh=_mesh,
    compiler_params=pltpu.CompilerParams(use_tc_tiling_on_sc=False, needs_layout_passes=False),
    scratch_types=[
        pltpu.VMEM_SHARED((NR, C), f32),
        pltpu.VMEM((NR, C), f32),
        pltpu.VMEM((NR,), jnp.int32),
        pltpu.VMEM((NBLK, K), jnp.int32),
        pltpu.VMEM((NR // 16, C), f32),
    ],
)


def _edge_body(g_hbm, src_hbm, dst_hbm, p_hbm, acc_s,
               buf0, buf1, sbig, dbig, sem0, sem1, zsem):
    cid = lax.axis_index("c")
    sid = lax.axis_index("s")

    wid = sid * 2 + cid

    pltpu.async_copy(src_hbm.at[pl.ds(wid * NBLK, NBLK)], sbig, sem0)
    pltpu.async_copy(dst_hbm.at[pl.ds(wid * NBLK, NBLK)], dbig, sem1)

    def zrow(r, _):
        for c4 in range(C // 16):
            buf0[r, pl.ds(c4 * 16, 16)] = jnp.zeros((16,), f32)
        return 0
    lax.fori_loop(0, K, zrow, 0)

    for k in range(RPT // K):
        pltpu.async_copy(buf0, acc_s.at[pl.ds(sid * RPT + k * K, K)], zsem)
    for k in range(RPT // K):
        pltpu.make_async_copy(
            buf0, acc_s.at[pl.ds(sid * RPT + k * K, K)], zsem).wait()
    pltpu.make_async_copy(src_hbm.at[pl.ds(wid * NBLK, NBLK)], sbig, sem0).wait()
    pltpu.make_async_copy(dst_hbm.at[pl.ds(wid * NBLK, NBLK)], dbig, sem1).wait()
    plsc.subcore_barrier()

    def start(b, bf, sem):
        pltpu.async_copy(g_hbm.at[sbig.at[b]], bf, sem)

    def drain(b, bf, sem):
        pltpu.make_async_copy(g_hbm.at[sbig.at[b]], bf, sem).wait()
        pltpu.sync_copy(bf, acc_s.at[dbig.at[b]], add=True)

    start(0, buf0, sem0)

    def pair(o, _):
        start(2 * o + 1, buf1, sem1)
        drain(2 * o, buf0, sem0)
        start(2 * o + 2, buf0, sem0)
        drain(2 * o + 1, buf1, sem1)
        return 0
    lax.fori_loop(0, NBLK // 2, pair, 0)
    drain(NBLK - 1, buf0, sem0)
    plsc.subcore_barrier()

    def s2v(k, bf, sem):
        pltpu.async_copy(acc_s.at[pl.ds(sid * RPT + k * K, K)], bf, sem)

    def s2v_wait(k, bf, sem):
        pltpu.make_async_copy(
            acc_s.at[pl.ds(sid * RPT + k * K, K)], bf, sem).wait()

    def v2h(k, bf, sem):
        pltpu.async_copy(bf, p_hbm.at[cid, pl.ds(sid * RPT + k * K, K)], sem)

    def v2h_wait(k, bf, sem):
        pltpu.make_async_copy(
            bf, p_hbm.at[cid, pl.ds(sid * RPT + k * K, K)], sem).wait()

    nwb = RPT // K
    bufs = [(buf0, sem0), (buf1, sem1)]
    s2v(0, *bufs[0])
    for k in range(nwb):
        cur = bufs[k % 2]
        oth = bufs[(k + 1) % 2]
        s2v_wait(k, *cur)
        if k >= 1:
            v2h_wait(k - 1, *oth)
        if k < nwb - 1:
            s2v(k + 1, *oth)
        v2h(k, *cur)
    v2h_wait(nwb - 1, *bufs[(nwb - 1) % 2])


_edge_call = pl.kernel(
    _edge_body,
    out_type=jax.ShapeDtypeStruct((2, NPAD, C), f32),
    mesh=_mesh,
    compiler_params=pltpu.CompilerParams(use_tc_tiling_on_sc=False, needs_layout_passes=False),
    scratch_types=[
        pltpu.VMEM_SHARED((NPAD, C), f32),
        pltpu.VMEM((K, C), f32),
        pltpu.VMEM((K, C), f32),
        pltpu.VMEM((NBLK, K), jnp.int32),
        pltpu.VMEM((NBLK, K), jnp.int32),
        pltpu.SemaphoreType.DMA,
        pltpu.SemaphoreType.DMA,
        pltpu.SemaphoreType.DMA,
    ],
)


NBLK2 = (E // K) // 16


def _fin_body(gf_hbm, src_hbm, dst_hbm, dinv_hbm, bf_hbm, o_hbm,
              acc_s, acc_l, gf_v, idt, sbig, dbig, cbuf, dv, obuf, bfv):
    cid = lax.axis_index("c")
    sid = lax.axis_index("s")
    _zero_2d(acc_l, NR)
    _fill_iota(idt, NR)
    rows = NR // 16
    pltpu.sync_copy(acc_l.at[pl.ds(0, rows)], acc_s.at[pl.ds(sid * rows, rows)])
    pltpu.sync_copy(gf_hbm, gf_v.at[pl.ds(0, N)])
    pltpu.sync_copy(src_hbm.at[pl.ds(sid * NBLK2, NBLK2)], sbig)
    pltpu.sync_copy(dst_hbm.at[pl.ds(sid * NBLK2, NBLK2)], dbig)
    plsc.subcore_barrier()

    def ebody(i, _):
        for j in range(K // 16):
            s16 = sbig[i, pl.ds(j * 16, 16)]
            d16 = dbig[i, pl.ds(j * 16, 16)]
            vals = plsc.load_gather(gf_v, [s16])
            row = lax.shift_right_logical(d16, 7)
            col = jnp.bitwise_and(d16, 127)
            plsc.addupdate_scatter(acc_l, [row, col], vals)
        return 0
    lax.fori_loop(0, NBLK2, ebody, 0)
    plsc.subcore_barrier()
    pltpu.sync_copy(acc_l, acc_s.at[idt], add=True)
    plsc.subcore_barrier()

    @pl.when(cid == 0)
    def _():
        pltpu.sync_copy(acc_s.at[pl.ds(sid * (NR // 16), NR // 16)], cbuf)
        pltpu.sync_copy(dinv_hbm.at[pl.ds(sid * RPT, RPT)], dv)
        pltpu.sync_copy(bf_hbm, bfv)
        b16 = bfv[...]

        def comb(j, _):
            row = lax.shift_right_logical(j, 3)
            col = jnp.bitwise_and(j, 7) * 16
            a16 = cbuf[row, pl.ds(col, 16)]
            g16 = gf_v[pl.ds(sid * RPT + j * 16, 16)]
            d16 = dv[pl.ds(j * 16, 16)]
            obuf[pl.ds(j * 16, 16)] = d16 * (a16 + g16) + b16
            return 0
        lax.fori_loop(0, RPT // 16, comb, 0)
        pltpu.sync_copy(obuf, o_hbm.at[pl.ds(sid * RPT, RPT)])


_fin_call = pl.kernel(
    _fin_body,
    out_type=jax.ShapeDtypeStruct((NPAD,), f32),
    mesh=_mesh,
    compiler_params=pltpu.CompilerParams(use_tc_tiling_on_sc=False, needs_layout_passes=False),
    scratch_types=[
        pltpu.VMEM_SHARED((NR, C), f32),
        pltpu.VMEM((NR, C), f32),
        pltpu.VMEM((NPAD,), f32),
        pltpu.VMEM((NR,), jnp.int32),
        pltpu.VMEM((NBLK2, K), jnp.int32),
        pltpu.VMEM((NBLK2, K), jnp.int32),
        pltpu.VMEM((NR // 16, C), f32),
        pltpu.VMEM((RPT,), f32),
        pltpu.VMEM((RPT,), f32),
        pltpu.VMEM((16,), f32),
    ],
)


def _tc1_body(x_ref, w_ref, d0_ref, d1_ref, g_ref, dinv_ref):
    dinv = lax.rsqrt(d0_ref[0] + d1_ref[0] + 1.0)
    g_ref[...] = jnp.dot(x_ref[...], w_ref[...],
                         preferred_element_type=f32) * dinv
    dinv_ref[...] = dinv


def _tc2_body(p0_ref, p1_ref, g_ref, dinv_ref, b_ref, w_ref, o_ref):
    dinv = dinv_ref[...]
    h = jnp.maximum(
        dinv * (p0_ref[0] + p1_ref[0] + g_ref[...]) + b_ref[...], 0.0)
    o_ref[...] = jnp.dot(h, w_ref[...], preferred_element_type=f32) * dinv


def _row_spec(w):
    return pl.BlockSpec((R, w), lambda i: (i, 0))


def _const_spec(h, w):
    return pl.BlockSpec((h, w), lambda i: (0, 0))


def _half_spec(c, w):
    return pl.BlockSpec((1, R, w), lambda i, c=c: (c, i, 0))


_GRID = N // R

_tc1_call = pl.pallas_call(
    _tc1_body,
    grid=(_GRID,),
    in_specs=[_row_spec(C), _const_spec(C, C), _half_spec(0, 1),
              _half_spec(1, 1)],
    out_specs=[_row_spec(C), _row_spec(1)],
    out_shape=[jax.ShapeDtypeStruct((N, C), f32),
               jax.ShapeDtypeStruct((NPAD, 1), f32)],
)


def _make_tc2(cout):
    return pl.pallas_call(
        _tc2_body,
        grid=(_GRID,),
        in_specs=[_half_spec(0, C), _half_spec(1, C), _row_spec(C),
                  _row_spec(1), _const_spec(1, C), _const_spec(C, cout)],
        out_specs=_row_spec(cout),
        out_shape=jax.ShapeDtypeStruct((N, cout), f32),
    )


_tc2_call = _make_tc2(C)
_tc3_call = _make_tc2(1)

@jax.jit
def kernel(x, edge_index, batch, W0, b0, W1, b1, Wf, bf):
    src = edge_index[0].reshape(E // K, K)
    dst = edge_index[1].reshape(E // K, K)

    d = _deg_call(dst).reshape(2, NPAD, 1)
    g0, dinv = _tc1_call(x, W0, d, d)
    p = _edge_call(g0, src, dst)
    g1 = _tc2_call(p, p, g0, dinv, b0.reshape(1, C), W1)
    q = _edge_call(g1, src, dst)
    gf = _tc3_call(q, q, g1, dinv, b1.reshape(1, C), Wf)
    outp = _fin_call(gf.reshape(N), src, dst, dinv.reshape(NPAD),
                     jnp.broadcast_to(bf, (16,)))
    return outp[:N].reshape(N, 1)


# speedup vs baseline: 1.3643x; 1.0107x over previous
import jax
import jax.numpy as jnp
from jax import lax
from jax.experimental import pallas as pl
from jax.experimental.pallas import tpu as pltpu
from jax.experimental.pallas import tpu_sc as plsc

N = 10000
E = 320000
C = 128
NPAD = 10240
RPT = 640
K = 80
R = 5000
EPW = E // 32
NBLK = EPW // 80

_mesh = plsc.VectorSubcoreMesh(core_axis_name="c", subcore_axis_name="s")
f32 = jnp.float32


def _fill_vec(ref, n, val):
    def body(j, _):
        ref[pl.ds(j * 16, 16)] = jnp.full((16,), val, f32)
        return 0
    lax.fori_loop(0, n // 16, body, 0)


NR = NPAD // C


def _zero_2d(ref, rows):
    def body(r, _):
        for c4 in range(C // 16):
            ref[r, pl.ds(c4 * 16, 16)] = jnp.zeros((16,), f32)
        return 0
    lax.fori_loop(0, rows, body, 0)


def _fill_iota(ref, n):
    def body(j, _):
        ref[pl.ds(j * 16, 16)] = jnp.arange(16, dtype=jnp.int32) + j * 16
        return 0
    lax.fori_loop(0, n // 16, body, 0)


def _drain_and_writeback(acc_l, acc_s, idt, out_hbm, cid, sid, wbuf):
    pltpu.sync_copy(acc_l, acc_s.at[idt], add=True)
    plsc.subcore_barrier()
    rows = NR // 16
    pltpu.sync_copy(acc_s.at[pl.ds(sid * rows, rows)], wbuf)
    pltpu.sync_copy(wbuf, out_hbm.at[cid, pl.ds(sid * rows, rows)])


def _deg_body(dst_hbm, d_hbm, acc_s, acc_l, idt, dbig, wbuf):
    cid = lax.axis_index("c")
    sid = lax.axis_index("s")
    _zero_2d(acc_l, NR)
    _fill_iota(idt, NR)
    rows = NR // 16
    pltpu.sync_copy(acc_l.at[pl.ds(0, rows)], acc_s.at[pl.ds(sid * rows, rows)])
    wid = sid * 2 + cid
    pltpu.sync_copy(dst_hbm.at[pl.ds(wid * NBLK, NBLK)], dbig)
    plsc.subcore_barrier()

    ones16 = jnp.ones((16,), f32)

    def ebody(i, _):
        for j in range(K // 16):
            d16 = dbig[i, pl.ds(j * 16, 16)]
            row = lax.shift_right_logical(d16, 7)
            col = jnp.bitwise_and(d16, 127)
            plsc.addupdate_scatter(acc_l, [row, col], ones16)
        return 0
    lax.fori_loop(0, NBLK, ebody, 0)
    plsc.subcore_barrier()
    _drain_and_writeback(acc_l, acc_s, idt, d_hbm, cid, sid, wbuf)


_deg_call = pl.kernel(
    _deg_body,
    out_type=jax.ShapeDtypeStruct((2, NR, C), f32),
    mesh=_mesh,
    compiler_params=pltpu.CompilerParams(use_tc_tiling_on_sc=False, needs_layout_passes=False),
    scratch_types=[
        pltpu.VMEM_SHARED((NR, C), f32),
        pltpu.VMEM((NR, C), f32),
        pltpu.VMEM((NR,), jnp.int32),
        pltpu.VMEM((NBLK, K), jnp.int32),
        pltpu.VMEM((NR // 16, C), f32),
    ],
)


def _edge_body(g_hbm, src_hbm, dst_hbm, p_hbm, acc_s,
               buf0, buf1, sbig, dbig, sem0, sem1, zsem):
    cid = lax.axis_index("c")
    sid = lax.axis_index("s")

    wid = sid * 2 + cid

    pltpu.async_copy(src_hbm.at[pl.ds(wid * NBLK, NBLK)], sbig, sem0)
    pltpu.async_copy(dst_hbm.at[pl.ds(wid * NBLK, NBLK)], dbig, sem1)

    def zrow(r, _):
        for c4 in range(C // 16):
            buf0[r, pl.ds(c4 * 16, 16)] = jnp.zeros((16,), f32)
        return 0
    lax.fori_loop(0, K, zrow, 0)

    for k in range(RPT // K):
        pltpu.async_copy(buf0, acc_s.at[pl.ds(sid * RPT + k * K, K)], zsem)
    for k in range(RPT // K):
        pltpu.make_async_copy(
            buf0, acc_s.at[pl.ds(sid * RPT + k * K, K)], zsem).wait()
    pltpu.make_async_copy(src_hbm.at[pl.ds(wid * NBLK, NBLK)], sbig, sem0).wait()
    pltpu.make_async_copy(dst_hbm.at[pl.ds(wid * NBLK, NBLK)], dbig, sem1).wait()
    plsc.subcore_barrier()

    def start(b, bf, sem):
        pltpu.async_copy(g_hbm.at[sbig.at[b]], bf, sem)

    def drain(b, bf, sem):
        pltpu.make_async_copy(g_hbm.at[sbig.at[b]], bf, sem).wait()
        pltpu.sync_copy(bf, acc_s.at[dbig.at[b]], add=True)

    start(0, buf0, sem0)

    def pair(o, _):
        start(2 * o + 1, buf1, sem1)
        drain(2 * o, buf0, sem0)
        start(2 * o + 2, buf0, sem0)
        drain(2 * o + 1, buf1, sem1)
        return 0
    lax.fori_loop(0, NBLK // 2, pair, 0)
    drain(NBLK - 1, buf0, sem0)
    plsc.subcore_barrier()

    def s2v(k, bf, sem):
        pltpu.async_copy(acc_s.at[pl.ds(sid * RPT + k * K, K)], bf, sem)

    def s2v_wait(k, bf, sem):
        pltpu.make_async_copy(
            acc_s.at[pl.ds(sid * RPT + k * K, K)], bf, sem).wait()

    def v2h(k, bf, sem):
        pltpu.async_copy(bf, p_hbm.at[cid, pl.ds(sid * RPT + k * K, K)], sem)

    def v2h_wait(k, bf, sem):
        pltpu.make_async_copy(
            bf, p_hbm.at[cid, pl.ds(sid * RPT + k * K, K)], sem).wait()

    nwb = RPT // K
    bufs = [(buf0, sem0), (buf1, sem1)]
    s2v(0, *bufs[0])
    for k in range(nwb):
        cur = bufs[k % 2]
        oth = bufs[(k + 1) % 2]
        s2v_wait(k, *cur)
        if k >= 1:
            v2h_wait(k - 1, *oth)
        if k < nwb - 1:
            s2v(k + 1, *oth)
        v2h(k, *cur)
    v2h_wait(nwb - 1, *bufs[(nwb - 1) % 2])


_edge_call = pl.kernel(
    _edge_body,
    out_type=jax.ShapeDtypeStruct((2, NPAD, C), f32),
    mesh=_mesh,
    compiler_params=pltpu.CompilerParams(use_tc_tiling_on_sc=False, needs_layout_passes=False),
    scratch_types=[
        pltpu.VMEM_SHARED((NPAD, C), f32),
        pltpu.VMEM((K, C), f32),
        pltpu.VMEM((K, C), f32),
        pltpu.VMEM((NBLK, K), jnp.int32),
        pltpu.VMEM((NBLK, K), jnp.int32),
        pltpu.SemaphoreType.DMA,
        pltpu.SemaphoreType.DMA,
        pltpu.SemaphoreType.DMA,
    ],
)


NBLK2 = (E // K) // 16


def _fin_body(gf_hbm, src_hbm, dst_hbm, dinv_hbm, bf_hbm, o_hbm,
              acc_s, acc_l, gf_v, idt, sbig, dbig, cbuf, dv, obuf, bfv):
    cid = lax.axis_index("c")
    sid = lax.axis_index("s")
    _zero_2d(acc_l, NR)
    _fill_iota(idt, NR)
    rows = NR // 16
    pltpu.sync_copy(acc_l.at[pl.ds(0, rows)], acc_s.at[pl.ds(sid * rows, rows)])
    pltpu.sync_copy(gf_hbm, gf_v.at[pl.ds(0, N)])
    pltpu.sync_copy(src_hbm.at[pl.ds(sid * NBLK2, NBLK2)], sbig)
    pltpu.sync_copy(dst_hbm.at[pl.ds(sid * NBLK2, NBLK2)], dbig)
    plsc.subcore_barrier()

    def ebody(i, _):
        for j in range(K // 16):
            s16 = sbig[i, pl.ds(j * 16, 16)]
            d16 = dbig[i, pl.ds(j * 16, 16)]
            vals = plsc.load_gather(gf_v, [s16])
            row = lax.shift_right_logical(d16, 7)
            col = jnp.bitwise_and(d16, 127)
            plsc.addupdate_scatter(acc_l, [row, col], vals)
        return 0
    lax.fori_loop(0, NBLK2, ebody, 0)
    plsc.subcore_barrier()
    pltpu.sync_copy(acc_l, acc_s.at[idt], add=True)
    plsc.subcore_barrier()

    @pl.when(cid == 0)
    def _():
        pltpu.sync_copy(acc_s.at[pl.ds(sid * (NR // 16), NR // 16)], cbuf)
        pltpu.sync_copy(dinv_hbm.at[pl.ds(sid * RPT, RPT)], dv)
        pltpu.sync_copy(bf_hbm, bfv)
        b16 = bfv[...]

        def comb(j, _):
            row = lax.shift_right_logical(j, 3)
            col = jnp.bitwise_and(j, 7) * 16
            a16 = cbuf[row, pl.ds(col, 16)]
            g16 = gf_v[pl.ds(sid * RPT + j * 16, 16)]
            d16 = dv[pl.ds(j * 16, 16)]
            obuf[pl.ds(j * 16, 16)] = d16 * (a16 + g16) + b16
            return 0
        lax.fori_loop(0, RPT // 16, comb, 0)
        pltpu.sync_copy(obuf, o_hbm.at[pl.ds(sid * RPT, RPT)])


_fin_call = pl.kernel(
    _fin_body,
    out_type=jax.ShapeDtypeStruct((NPAD,), f32),
    mesh=_mesh,
    compiler_params=pltpu.CompilerParams(use_tc_tiling_on_sc=False, needs_layout_passes=False),
    scratch_types=[
        pltpu.VMEM_SHARED((NR, C), f32),
        pltpu.VMEM((NR, C), f32),
        pltpu.VMEM((NPAD,), f32),
        pltpu.VMEM((NR,), jnp.int32),
        pltpu.VMEM((NBLK2, K), jnp.int32),
        pltpu.VMEM((NBLK2, K), jnp.int32),
        pltpu.VMEM((NR // 16, C), f32),
        pltpu.VMEM((RPT,), f32),
        pltpu.VMEM((RPT,), f32),
        pltpu.VMEM((16,), f32),
    ],
)


def _tc1_body(x_ref, w_ref, d0_ref, d1_ref, g_ref, dinv_ref):
    dinv = lax.rsqrt(d0_ref[0] + d1_ref[0] + 1.0)
    g_ref[...] = jnp.dot(x_ref[...], w_ref[...],
                         preferred_element_type=f32) * dinv
    dinv_ref[...] = dinv


def _tc2_body(p0_ref, p1_ref, g_ref, dinv_ref, b_ref, w_ref, o_ref):
    dinv = dinv_ref[...]
    h = jnp.maximum(
        dinv * (p0_ref[0] + p1_ref[0] + g_ref[...]) + b_ref[...], 0.0)
    o_ref[...] = jnp.dot(h, w_ref[...], preferred_element_type=f32) * dinv


def _row_spec(w):
    return pl.BlockSpec((R, w), lambda i: (i, 0))


def _const_spec(h, w):
    return pl.BlockSpec((h, w), lambda i: (0, 0))


def _half_spec(c, w):
    return pl.BlockSpec((1, R, w), lambda i, c=c: (c, i, 0))


_GRID = N // R

_tc1_call = pl.pallas_call(
    _tc1_body,
    grid=(_GRID,),
    in_specs=[_row_spec(C), _const_spec(C, C), _half_spec(0, 1),
              _half_spec(1, 1)],
    out_specs=[_row_spec(C), _row_spec(1)],
    out_shape=[jax.ShapeDtypeStruct((N, C), f32),
               jax.ShapeDtypeStruct((NPAD, 1), f32)],
)


def _make_tc2(cout):
    return pl.pallas_call(
        _tc2_body,
        grid=(_GRID,),
        in_specs=[_half_spec(0, C), _half_spec(1, C), _row_spec(C),
                  _row_spec(1), _const_spec(1, C), _const_spec(C, cout)],
        out_specs=_row_spec(cout),
        out_shape=jax.ShapeDtypeStruct((N, cout), f32),
    )


_tc2_call = _make_tc2(C)
_tc3_call = _make_tc2(1)

@jax.jit
def kernel(x, edge_index, batch, W0, b0, W1, b1, Wf, bf):
    src = edge_index[0].reshape(E // K, K)
    dst = edge_index[1].reshape(E // K, K)

    d = _deg_call(dst).reshape(2, NPAD, 1)
    g0, dinv = _tc1_call(x, W0, d, d)
    p = _edge_call(g0, src, dst)
    g1 = _tc2_call(p, p, g0, dinv, b0.reshape(1, C), W1)
    q = _edge_call(g1, src, dst)
    gf = _tc3_call(q, q, g1, dinv, b1.reshape(1, C), Wf)
    outp = _fin_call(gf.reshape(N), src, dst, dinv.reshape(NPAD),
                     jnp.broadcast_to(bf, (16,)))
    return outp[:N].reshape(N, 1)
